# baseline (speedup 1.0000x reference)
import functools

import jax
import jax.numpy as jnp
from jax import lax
from jax.experimental import pallas as pl
from jax.experimental.pallas import tpu as pltpu
from jax.experimental.pallas import tpu_sc as plsc

N = 10000
E = 320000
D = 128
NC = 2
NS = 16
EPT = E // (NC * NS)
NPAD = 10112
RPT = NPAD // NS
DGR = 128

K_MP = 96
K_DG = 80

_MESH = plsc.VectorSubcoreMesh(core_axis_name="c", subcore_axis_name="s")


def _nchunks(k):
    n = -(-EPT // k)
    return n + (n % 2)


def _make_gs_pass(k, acc_rows, rpt):
    nch = _nchunks(k)

    @functools.partial(
        pl.kernel,
        out_type=jax.ShapeDtypeStruct((NC, acc_rows, D), jnp.float32),
        mesh=_MESH,
        scratch_types=[
            pltpu.VMEM((2, 2, k), jnp.int32),
            pltpu.VMEM((k, D), jnp.float32),
            pltpu.VMEM((k, D), jnp.float32),
            pltpu.VMEM_SHARED((acc_rows, D), jnp.float32),
            pltpu.SemaphoreType.DMA,
            pltpu.SemaphoreType.DMA,
            pltpu.SemaphoreType.DMA,
        ],
    )
    def gs_pass(table_hbm, ei_hbm, z_hbm, out_hbm,
                idxv, buf0, buf1, acc, semi, sem0, sem1):
        c = lax.axis_index("c")
        s = lax.axis_index("s")
        pltpu.sync_copy(z_hbm.at[pl.ds(s * rpt, rpt)], acc.at[pl.ds(s * rpt, rpt)])
        plsc.subcore_barrier()

        pltpu.sync_copy(ei_hbm.at[c, s, 0], idxv.at[0])
        pltpu.sync_copy(ei_hbm.at[c, s, 1], idxv.at[1])
        pltpu.sync_copy(table_hbm.at[idxv.at[0, 0]], buf0)

        def body(i, carry):
            a = 2 * i
            g1 = pltpu.async_copy(table_hbm.at[idxv.at[1, 0]], buf1, sem1)
            pltpu.sync_copy(buf0, acc.at[idxv.at[0, 1]], add=True)
            i0 = pltpu.async_copy(ei_hbm.at[c, s, a + 2], idxv.at[0], semi)
            g1.wait()
            i0.wait()
            g0 = pltpu.async_copy(table_hbm.at[idxv.at[0, 0]], buf0, sem0)
            pltpu.sync_copy(buf1, acc.at[idxv.at[1, 1]], add=True)
            i1 = pltpu.async_copy(ei_hbm.at[c, s, a + 3], idxv.at[1], semi)
            g0.wait()
            i1.wait()
            return carry

        lax.fori_loop(0, nch // 2 - 1, body, 0)
        g1 = pltpu.async_copy(table_hbm.at[idxv.at[1, 0]], buf1, sem1)
        pltpu.sync_copy(buf0, acc.at[idxv.at[0, 1]], add=True)
        g1.wait()
        pltpu.sync_copy(buf1, acc.at[idxv.at[1, 1]], add=True)

        plsc.subcore_barrier()
        pltpu.sync_copy(acc.at[pl.ds(s * rpt, rpt)], out_hbm.at[c, pl.ds(s * rpt, rpt)])

    return gs_pass


_deg_pass = _make_gs_pass(K_DG, DGR, DGR // NS)
_mp_pass = _make_gs_pass(K_MP, NPAD, RPT)


_R = 1264


def _ln_mm_body(x_ref, w_ref, g_ref, bt_ref, dc_ref, y_ref):
    xb = x_ref[...]
    mu = jnp.mean(xb, axis=-1, keepdims=True)
    var = jnp.mean((xb - mu) ** 2, axis=-1, keepdims=True)
    xln = (xb - mu) * lax.rsqrt(var + 1e-5) * g_ref[...] + bt_ref[...]
    deg = dc_ref[0] + dc_ref[1] + 1.0
    y_ref[...] = jnp.dot(xln, w_ref[...], preferred_element_type=jnp.float32) * lax.rsqrt(deg)


_ln_mm = pl.pallas_call(
    _ln_mm_body,
    grid=(NPAD // _R,),
    in_specs=[
        pl.BlockSpec((_R, D), lambda i: (i, 0)),
        pl.BlockSpec((D, D), lambda i: (0, 0)),
        pl.BlockSpec((1, D), lambda i: (0, 0)),
        pl.BlockSpec((1, D), lambda i: (0, 0)),
        pl.BlockSpec((NC, _R, 1), lambda i: (0, i, 0)),
    ],
    out_specs=pl.BlockSpec((_R, D), lambda i: (i, 0)),
    out_shape=jax.ShapeDtypeStruct((NPAD, D), jnp.float32),
)


def _final_body(acc_ref, y_ref, x_ref, b_ref, dc_ref, o_ref):
    deg = dc_ref[0] + dc_ref[1] + 1.0
    dinv = lax.rsqrt(deg)
    agg = acc_ref[0] + acc_ref[1] + y_ref[...]
    o_ref[...] = jnp.maximum(dinv * agg + b_ref[...], 0.0) + x_ref[...]


_final = pl.pallas_call(
    _final_body,
    grid=(NPAD // _R,),
    in_specs=[
        pl.BlockSpec((NC, _R, D), lambda i: (0, i, 0)),
        pl.BlockSpec((_R, D), lambda i: (i, 0)),
        pl.BlockSpec((_R, D), lambda i: (i, 0)),
        pl.BlockSpec((1, D), lambda i: (0, 0)),
        pl.BlockSpec((NC, _R, 1), lambda i: (0, i, 0)),
    ],
    out_specs=pl.BlockSpec((_R, D), lambda i: (i, 0)),
    out_shape=jax.ShapeDtypeStruct((NPAD, D), jnp.float32),
)


def _chunked_pair(g_idx, s_idx, k, g_fill, s_fill):
    nch = _nchunks(k)
    pad = nch * k - EPT
    g = jnp.pad(g_idx.reshape(NC * NS, EPT), ((0, 0), (0, pad)),
                constant_values=g_fill).reshape(NC, NS, nch, 1, k)
    s = jnp.pad(s_idx.reshape(NC * NS, EPT), ((0, 0), (0, pad)),
                constant_values=s_fill).reshape(NC, NS, nch, 1, k)
    return jnp.concatenate([g, s], axis=3)


def kernel(x, edge_attr, edge_index, W, b, gamma, beta):
    src, dst = edge_index[0], edge_index[1]
    ei_dg = _chunked_pair(dst & (D - 1), dst >> 7, K_DG, 0, DGR - 1)
    ei_mp = _chunked_pair(src, dst, K_MP, 0, N)
    eye = jnp.eye(D, dtype=jnp.float32)
    z_dg = jnp.zeros((DGR, D), jnp.float32)
    z_mp = jnp.zeros((NPAD, D), jnp.float32)
    x_pad = jnp.concatenate([x, jnp.zeros((NPAD - N, D), x.dtype)], axis=0)

    dcnt = _deg_pass(eye, ei_dg, z_dg)
    dcnt3 = dcnt.reshape(NC, DGR * D, 1)
    y = _ln_mm(x_pad, W, gamma.reshape(1, D), beta.reshape(1, D), dcnt3)
    acc = _mp_pass(y, ei_mp, z_mp)
    x_out = _final(acc, y, x_pad, b.reshape(1, D), dcnt3)
    return (x_out[:N], edge_attr)


# speedup vs baseline: 1.2328x; 1.2328x over previous
import functools

import jax
import jax.numpy as jnp
from jax import lax
from jax.experimental import pallas as pl
from jax.experimental.pallas import tpu as pltpu
from jax.experimental.pallas import tpu_sc as plsc

N = 10000
E = 320000
D = 128
NC = 2
NS = 16
EPT = E // (NC * NS)
NPAD = 10112
RPT = NPAD // NS
DGR = 128

K_MP, NB_MP = 176, 3
K_DG, NB_DG = 144, 5

_MESH = plsc.VectorSubcoreMesh(core_axis_name="c", subcore_axis_name="s")


def _nchunks(k, nb):
    n = -(-EPT // k)
    return -(-n // nb) * nb


def _make_gs_pass(k, nb, acc_rows, rpt):
    nblk = _nchunks(k, nb) // nb

    @functools.partial(
        pl.kernel,
        out_type=jax.ShapeDtypeStruct((NC, acc_rows, D), jnp.float32),
        mesh=_MESH,
        scratch_types=[
            pltpu.VMEM((nb * 2 * k,), jnp.int32),
            pltpu.VMEM((k, D), jnp.float32),
            pltpu.VMEM_SHARED((acc_rows, D), jnp.float32),
        ],
    )
    def gs_pass(table_hbm, ei_hbm, z_hbm, out_hbm, idxv, buf, acc):
        c = lax.axis_index("c")
        s = lax.axis_index("s")
        pltpu.sync_copy(z_hbm.at[pl.ds(s * rpt, rpt)], acc.at[pl.ds(s * rpt, rpt)])
        plsc.subcore_barrier()

        def body(blk, carry):
            pltpu.sync_copy(ei_hbm.at[c, s, blk], idxv)
            for jj in range(nb):
                gsl = idxv.at[pl.ds((2 * jj) * k, k)]
                ssl = idxv.at[pl.ds((2 * jj + 1) * k, k)]
                pltpu.sync_copy(table_hbm.at[gsl], buf)
                pltpu.sync_copy(buf, acc.at[ssl], add=True)
            return carry

        lax.fori_loop(0, nblk, body, 0)

        plsc.subcore_barrier()
        pltpu.sync_copy(acc.at[pl.ds(s * rpt, rpt)], out_hbm.at[c, pl.ds(s * rpt, rpt)])

    return gs_pass


_deg_pass = _make_gs_pass(K_DG, NB_DG, DGR, DGR // NS)
_mp_pass = _make_gs_pass(K_MP, NB_MP, NPAD, RPT)


_R = 1264


def _ln_mm_body(x_ref, w_ref, g_ref, bt_ref, dc_ref, y_ref):
    xb = x_ref[...]
    mu = jnp.mean(xb, axis=-1, keepdims=True)
    var = jnp.mean((xb - mu) ** 2, axis=-1, keepdims=True)
    xln = (xb - mu) * lax.rsqrt(var + 1e-5) * g_ref[...] + bt_ref[...]
    deg = dc_ref[0] + dc_ref[1] + 1.0
    y_ref[...] = jnp.dot(xln, w_ref[...], preferred_element_type=jnp.float32) * lax.rsqrt(deg)


_ln_mm = pl.pallas_call(
    _ln_mm_body,
    grid=(NPAD // _R,),
    in_specs=[
        pl.BlockSpec((_R, D), lambda i: (i, 0)),
        pl.BlockSpec((D, D), lambda i: (0, 0)),
        pl.BlockSpec((1, D), lambda i: (0, 0)),
        pl.BlockSpec((1, D), lambda i: (0, 0)),
        pl.BlockSpec((NC, _R, 1), lambda i: (0, i, 0)),
    ],
    out_specs=pl.BlockSpec((_R, D), lambda i: (i, 0)),
    out_shape=jax.ShapeDtypeStruct((NPAD, D), jnp.float32),
)


def _final_body(acc_ref, y_ref, x_ref, b_ref, dc_ref, o_ref):
    deg = dc_ref[0] + dc_ref[1] + 1.0
    dinv = lax.rsqrt(deg)
    agg = acc_ref[0] + acc_ref[1] + y_ref[...]
    o_ref[...] = jnp.maximum(dinv * agg + b_ref[...], 0.0) + x_ref[...]


_final = pl.pallas_call(
    _final_body,
    grid=(NPAD // _R,),
    in_specs=[
        pl.BlockSpec((NC, _R, D), lambda i: (0, i, 0)),
        pl.BlockSpec((_R, D), lambda i: (i, 0)),
        pl.BlockSpec((_R, D), lambda i: (i, 0)),
        pl.BlockSpec((1, D), lambda i: (0, 0)),
        pl.BlockSpec((NC, _R, 1), lambda i: (0, i, 0)),
    ],
    out_specs=pl.BlockSpec((_R, D), lambda i: (i, 0)),
    out_shape=jax.ShapeDtypeStruct((NPAD, D), jnp.float32),
)


def _chunked_pair(g_idx, s_idx, k, nb, g_fill, s_fill):
    nch = _nchunks(k, nb)
    pad = nch * k - EPT
    g = jnp.pad(g_idx.reshape(NC * NS, EPT), ((0, 0), (0, pad)),
                constant_values=g_fill).reshape(NC, NS, nch, 1, k)
    s = jnp.pad(s_idx.reshape(NC * NS, EPT), ((0, 0), (0, pad)),
                constant_values=s_fill).reshape(NC, NS, nch, 1, k)
    gs = jnp.concatenate([g, s], axis=3)
    return gs.reshape(NC, NS, nch // nb, nb * 2 * k)


def kernel(x, edge_attr, edge_index, W, b, gamma, beta):
    src, dst = edge_index[0], edge_index[1]
    ei_dg = _chunked_pair(dst & (D - 1), dst >> 7, K_DG, NB_DG, 0, DGR - 1)
    ei_mp = _chunked_pair(src, dst, K_MP, NB_MP, 0, N)
    eye = jnp.eye(D, dtype=jnp.float32)
    z_dg = jnp.zeros((DGR, D), jnp.float32)
    z_mp = jnp.zeros((NPAD, D), jnp.float32)
    x_pad = jnp.concatenate([x, jnp.zeros((NPAD - N, D), x.dtype)], axis=0)

    dcnt = _deg_pass(eye, ei_dg, z_dg)
    dcnt3 = dcnt.reshape(NC, DGR * D, 1)
    y = _ln_mm(x_pad, W, gamma.reshape(1, D), beta.reshape(1, D), dcnt3)
    acc = _mp_pass(y, ei_mp, z_mp)
    x_out = _final(acc, y, x_pad, b.reshape(1, D), dcnt3)
    return (x_out[:N], edge_attr)
